# baseline (device time: 102578 ns/iter reference)
import jax
import jax.numpy as jnp
from jax import lax
from jax.experimental import pallas as pl
from jax.experimental.pallas import tpu as pltpu

N_DEV = 4
SCALE = 0.08838834764831843
WINDOW = 128


def _ring_allreduce(partial):
    M, N = partial.shape
    CH = M // N_DEV

    def body(p_ref, out_ref, rs_send, rs_recv, ag, send_sems, recv_sems):
        my = lax.axis_index("i")
        left = lax.rem(my + N_DEV - 1, N_DEV)
        right = lax.rem(my + 1, N_DEV)

        barrier_sem = pltpu.get_barrier_semaphore()
        for nbr in (left, right):
            pl.semaphore_signal(
                barrier_sem, inc=1,
                device_id=(nbr,), device_id_type=pl.DeviceIdType.MESH,
            )
        pl.semaphore_wait(barrier_sem, 2)

        def my_chunk(c):
            return p_ref[pl.ds(c * CH, CH), :].astype(jnp.float32)

        for s in range(N_DEV - 1):
            c_send = lax.rem(my - s + 2 * N_DEV, N_DEV)
            if s == 0:
                rs_send[0, :, :] = p_ref[pl.ds(c_send * CH, CH), :]
            else:
                rs_send[s, :, :] = (
                    rs_recv[s - 1, :, :].astype(jnp.float32) + my_chunk(c_send)
                ).astype(jnp.bfloat16)
            rdma = pltpu.make_async_remote_copy(
                src_ref=rs_send.at[s],
                dst_ref=rs_recv.at[s],
                send_sem=send_sems.at[s],
                recv_sem=recv_sems.at[s],
                device_id=(right,),
                device_id_type=pl.DeviceIdType.MESH,
            )
            rdma.start()
            rdma.wait()

        red_c = lax.rem(my + 1, N_DEV)
        reduced = rs_recv[N_DEV - 2, :, :].astype(jnp.float32) + my_chunk(red_c)
        out_ref[pl.ds(red_c * CH, CH), :] = reduced
        ag[0, :, :] = reduced.astype(jnp.bfloat16)

        for t in range(N_DEV - 1):
            rdma = pltpu.make_async_remote_copy(
                src_ref=ag.at[t],
                dst_ref=ag.at[t + 1],
                send_sem=send_sems.at[N_DEV - 1 + t],
                recv_sem=recv_sems.at[N_DEV - 1 + t],
                device_id=(right,),
                device_id_type=pl.DeviceIdType.MESH,
            )
            rdma.start()
            rdma.wait()
            c = lax.rem(my - t + N_DEV, N_DEV)
            out_ref[pl.ds(c * CH, CH), :] = ag[t + 1, :, :].astype(jnp.float32)

    return pl.pallas_call(
        body,
        out_shape=jax.ShapeDtypeStruct((M, N), jnp.float32),
        in_specs=[pl.BlockSpec(memory_space=pltpu.VMEM)],
        out_specs=pl.BlockSpec(memory_space=pltpu.VMEM),
        scratch_shapes=[
            pltpu.VMEM((N_DEV - 1, CH, N), jnp.bfloat16),
            pltpu.VMEM((N_DEV - 1, CH, N), jnp.bfloat16),
            pltpu.VMEM((N_DEV, CH, N), jnp.bfloat16),
            pltpu.SemaphoreType.DMA((2 * (N_DEV - 1),)),
            pltpu.SemaphoreType.DMA((2 * (N_DEV - 1),)),
        ],
        compiler_params=pltpu.CompilerParams(collective_id=0),
    )(partial)


def kernel(x, Wq, K_ext, V_ext, Wo):
    my = lax.axis_index("i")
    B, Sq, D = x.shape
    _, Skv, Hl, Dh = K_ext.shape
    hd = Hl * Dh
    start = my * hd

    xb = x[0].astype(jnp.bfloat16)
    Wq_l = lax.dynamic_slice_in_dim(Wq, start, hd, axis=1).astype(jnp.bfloat16)
    Wo_l = lax.dynamic_slice_in_dim(Wo, start, hd, axis=0).astype(jnp.bfloat16)
    K = K_ext[0].astype(jnp.bfloat16)
    V = V_ext[0].astype(jnp.bfloat16)

    Q = jnp.dot(xb, Wq_l, preferred_element_type=jnp.float32)
    Q = Q.astype(jnp.bfloat16).reshape(Sq, Hl, Dh)

    BQ = 128
    nb = Sq // BQ
    Qb = Q.reshape(nb, BQ, Hl, Dh)
    pad = [(BQ, BQ), (0, 0), (0, 0)]
    Kp = jnp.pad(K, pad).reshape(nb + 2, BQ, Hl, Dh)
    Vp = jnp.pad(V, pad).reshape(nb + 2, BQ, Hl, Dh)
    K_band = jnp.concatenate([Kp[0:nb], Kp[1:nb + 1], Kp[2:nb + 2]], axis=1)
    V_band = jnp.concatenate([Vp[0:nb], Vp[1:nb + 1], Vp[2:nb + 2]], axis=1)

    scores = jnp.einsum(
        "bqhd,bkhd->bhqk", Qb, K_band, preferred_element_type=jnp.float32
    ) * SCALE
    b_idx = jnp.arange(nb)[:, None, None]
    qi = jnp.arange(BQ)[None, :, None]
    kj = jnp.arange(3 * BQ)[None, None, :]
    kg = (b_idx - 1) * BQ + kj
    mask = (jnp.abs(qi + BQ - kj) <= WINDOW) & (kg >= 0) & (kg < Skv)
    scores = jnp.where(mask[:, None, :, :], scores, -1e9)
    w = jax.nn.softmax(scores, axis=-1)

    ctx = jnp.einsum(
        "bhqk,bkhd->bqhd", w.astype(jnp.bfloat16), V_band,
        preferred_element_type=jnp.float32,
    ).reshape(Sq, hd)
    part = jnp.dot(
        ctx.astype(jnp.bfloat16), Wo_l, preferred_element_type=jnp.float32
    ).astype(jnp.bfloat16)

    out = _ring_allreduce(part)
    return out[None, :, :]


# device time: 74544 ns/iter; 1.3761x vs baseline; 1.3761x over previous
import jax
import jax.numpy as jnp
from jax import lax
from jax.experimental import pallas as pl
from jax.experimental.pallas import tpu as pltpu

N_DEV = 4
SCALE = 0.08838834764831843
WINDOW = 128


def _ring_allreduce(partial):
    M, N = partial.shape
    HC = N // 2
    HR = M // 2
    QR = M // 4

    def body(p_ref, out_ref,
             s1_send, s1_recv, s2_send, s2_recv,
             s3_send, s3_recv, s4_send, s4_recv,
             qtr_f32, send_sems, recv_sems):
        p = lax.axis_index("i")
        q1 = jnp.bitwise_xor(p, 1)
        q2 = 3 - p

        barrier_sem = pltpu.get_barrier_semaphore()
        for nbr in (q1, q2):
            pl.semaphore_signal(
                barrier_sem, inc=1,
                device_id=(nbr,), device_id_type=pl.DeviceIdType.MESH,
            )
        pl.semaphore_wait(barrier_sem, 2)

        h_own = [jnp.where((p == 1) | (p == 2), 1, 0), jnp.where(p >= 2, 1, 0)]
        q2nd = [jnp.where(p >= 2, 1, 0), lax.rem(p, 2)]
        partners = [(q1, q2), (q2, q1)]

        def exchange(src, dst, sem_idx, tgt):
            rdma = pltpu.make_async_remote_copy(
                src_ref=src, dst_ref=dst,
                send_sem=send_sems.at[sem_idx],
                recv_sem=recv_sems.at[sem_idx],
                device_id=(tgt,), device_id_type=pl.DeviceIdType.MESH,
            )
            rdma.start()
            return rdma

        rds = []
        for s in range(2):
            c0 = s * HC
            s1_send[s, :, :] = p_ref[pl.ds((1 - h_own[s]) * HR, HR),
                                     pl.ds(c0, HC)]
            rds.append(exchange(s1_send.at[s], s1_recv.at[s], s, partners[s][0]))
        for r in rds:
            r.wait()

        rds = []
        for s in range(2):
            c0 = s * HC
            half0 = h_own[s] * HR
            off_other = (1 - q2nd[s]) * QR
            off_own = q2nd[s] * QR
            s2_send[s, :, :] = (
                p_ref[pl.ds(half0 + off_other, QR), pl.ds(c0, HC)]
                .astype(jnp.float32)
                + s1_recv[s, pl.ds(off_other, QR), :].astype(jnp.float32)
            ).astype(jnp.bfloat16)
            qtr_f32[s, :, :] = (
                p_ref[pl.ds(half0 + off_own, QR), pl.ds(c0, HC)]
                .astype(jnp.float32)
                + s1_recv[s, pl.ds(off_own, QR), :].astype(jnp.float32)
            )
            rds.append(exchange(s2_send.at[s], s2_recv.at[s], 2 + s,
                                partners[s][1]))
        for r in rds:
            r.wait()

        rds = []
        for s in range(2):
            c0 = s * HC
            rq = 2 * h_own[s] + q2nd[s]
            reduced = qtr_f32[s, :, :] + s2_recv[s, :, :].astype(jnp.float32)
            out_ref[pl.ds(rq * QR, QR), pl.ds(c0, HC)] = reduced
            s3_send[s, :, :] = reduced.astype(jnp.bfloat16)
            rds.append(exchange(s3_send.at[s], s3_recv.at[s], 4 + s,
                                partners[s][1]))
        for r in rds:
            r.wait()

        rds = []
        for s in range(2):
            c0 = s * HC
            rq2 = 2 * h_own[s] + (1 - q2nd[s])
            out_ref[pl.ds(rq2 * QR, QR), pl.ds(c0, HC)] = (
                s3_recv[s, :, :].astype(jnp.float32))
            s4_send[s, pl.ds(q2nd[s] * QR, QR), :] = s3_send[s, :, :]
            s4_send[s, pl.ds((1 - q2nd[s]) * QR, QR), :] = s3_recv[s, :, :]
            rds.append(exchange(s4_send.at[s], s4_recv.at[s], 6 + s,
                                partners[s][0]))
        for r in rds:
            r.wait()
        for s in range(2):
            c0 = s * HC
            out_ref[pl.ds((1 - h_own[s]) * HR, HR), pl.ds(c0, HC)] = (
                s4_recv[s, :, :].astype(jnp.float32))

    return pl.pallas_call(
        body,
        out_shape=jax.ShapeDtypeStruct((M, N), jnp.float32),
        in_specs=[pl.BlockSpec(memory_space=pltpu.VMEM)],
        out_specs=pl.BlockSpec(memory_space=pltpu.VMEM),
        scratch_shapes=[
            pltpu.VMEM((2, HR, HC), jnp.bfloat16),
            pltpu.VMEM((2, HR, HC), jnp.bfloat16),
            pltpu.VMEM((2, QR, HC), jnp.bfloat16),
            pltpu.VMEM((2, QR, HC), jnp.bfloat16),
            pltpu.VMEM((2, QR, HC), jnp.bfloat16),
            pltpu.VMEM((2, QR, HC), jnp.bfloat16),
            pltpu.VMEM((2, HR, HC), jnp.bfloat16),
            pltpu.VMEM((2, HR, HC), jnp.bfloat16),
            pltpu.VMEM((2, QR, HC), jnp.float32),
            pltpu.SemaphoreType.DMA((8,)),
            pltpu.SemaphoreType.DMA((8,)),
        ],
        compiler_params=pltpu.CompilerParams(collective_id=0),
    )(partial)


def kernel(x, Wq, K_ext, V_ext, Wo):
    my = lax.axis_index("i")
    B, Sq, D = x.shape
    _, Skv, Hl, Dh = K_ext.shape
    hd = Hl * Dh
    start = my * hd

    xb = x[0].astype(jnp.bfloat16)
    Wq_l = lax.dynamic_slice_in_dim(Wq, start, hd, axis=1).astype(jnp.bfloat16)
    Wo_l = lax.dynamic_slice_in_dim(Wo, start, hd, axis=0).astype(jnp.bfloat16)
    K = K_ext[0].astype(jnp.bfloat16)
    V = V_ext[0].astype(jnp.bfloat16)

    Q = jnp.dot(xb, Wq_l, preferred_element_type=jnp.bfloat16)

    BQ = 128
    nb = Sq // BQ
    Qb = Q.reshape(nb, BQ, Hl, Dh)
    Kb = K.reshape(nb, BQ, Hl, Dh)
    Vb = V.reshape(nb, BQ, Hl, Dh)
    K_sub = jnp.roll(Kb, 1, axis=0)
    K_sup = jnp.roll(Kb, -1, axis=0)

    def qk(q, k):
        return jnp.einsum(
            "bqhd,bkhd->bhqk", q, k, preferred_element_type=jnp.float32
        ) * SCALE

    s_diag = qk(Qb, Kb)
    s_sub = qk(Qb, K_sub)
    s_sup = qk(Qb, K_sup)

    b_idx = jnp.arange(nb)[:, None, None, None]
    qi = jnp.arange(BQ)[None, None, :, None]
    kj = jnp.arange(BQ)[None, None, None, :]
    s_sub = jnp.where((kj >= qi) & (b_idx >= 1), s_sub, -1e9)
    s_sup = jnp.where((kj <= qi) & (b_idx <= nb - 2), s_sup, -1e9)

    m = jnp.maximum(
        s_diag.max(-1), jnp.maximum(s_sub.max(-1), s_sup.max(-1))
    )[..., None]
    e_diag = jnp.exp(s_diag - m)
    e_sub = jnp.exp(s_sub - m)
    e_sup = jnp.exp(s_sup - m)
    denom = (
        e_diag.sum(-1) + e_sub.sum(-1) + e_sup.sum(-1)
    )[..., None]

    def pv(e, v):
        return jnp.einsum(
            "bhqk,bkhd->bqhd", (e / denom).astype(jnp.bfloat16), v,
            preferred_element_type=jnp.float32,
        )

    ctx = (pv(e_diag, Vb) + pv(e_sub, jnp.roll(Vb, 1, axis=0))
           + pv(e_sup, jnp.roll(Vb, -1, axis=0)))
    ctx = ctx.astype(jnp.bfloat16).reshape(Sq, hd)
    part = jnp.dot(ctx, Wo_l, preferred_element_type=jnp.bfloat16)

    out = _ring_allreduce(part)
    return out[None, :, :]


# device time: 67236 ns/iter; 1.5256x vs baseline; 1.1087x over previous
import jax
import jax.numpy as jnp
from jax import lax
from jax.experimental import pallas as pl
from jax.experimental.pallas import tpu as pltpu

N_DEV = 4
SCALE = 0.08838834764831843
WINDOW = 128
BQ = 128


def kernel(x, Wq, K_ext, V_ext, Wo):
    my = lax.axis_index("i")
    B, Sq, D = x.shape
    _, Skv, Hl, Dh = K_ext.shape
    hd = Hl * Dh
    start = my * hd

    xb = x[0].astype(jnp.bfloat16)
    Wq_l = lax.dynamic_slice_in_dim(Wq, start, hd, axis=1).astype(jnp.bfloat16)
    Wo_l = lax.dynamic_slice_in_dim(Wo, start, hd, axis=0).astype(jnp.bfloat16)
    Kh = jnp.transpose(K_ext[0].astype(jnp.bfloat16), (1, 0, 2))
    Vh = jnp.transpose(V_ext[0].astype(jnp.bfloat16), (1, 0, 2))

    M, N = Sq, D
    HC = N // 2
    HR = M // 2
    QR = M // 4
    nb = Sq // BQ

    def body(x_ref, wq_ref, k_ref, v_ref, wo_ref, out_ref,
             q_buf, ctx_buf, part_own,
             s1_send, s1_recv, s2_send, s2_recv,
             s3_send, s3_recv, s4_send, s4_recv,
             qtr_f32, send_sems, recv_sems):
        p = lax.axis_index("i")
        q1 = jnp.bitwise_xor(p, 1)
        q2 = 3 - p

        barrier_sem = pltpu.get_barrier_semaphore()
        for nbr in (q1, q2):
            pl.semaphore_signal(
                barrier_sem, inc=1,
                device_id=(nbr,), device_id_type=pl.DeviceIdType.MESH,
            )
        pl.semaphore_wait(barrier_sem, 2)

        q_buf[:, :] = jnp.dot(
            x_ref[:, :], wq_ref[:, :], preferred_element_type=jnp.float32
        ).astype(jnp.bfloat16)

        for h in range(Hl):
            for b in range(nb):
                lo = max(0, b * BQ - WINDOW)
                hi = min(Skv, b * BQ + BQ + WINDOW)
                w = hi - lo
                qb = q_buf[pl.ds(b * BQ, BQ), pl.ds(h * Dh, Dh)]
                kwin = k_ref[h, pl.ds(lo, w), :]
                s = lax.dot_general(
                    qb, kwin, (((1,), (1,)), ((), ())),
                    preferred_element_type=jnp.float32,
                ) * SCALE
                iq = lax.broadcasted_iota(jnp.int32, (BQ, w), 0)
                ik = lax.broadcasted_iota(jnp.int32, (BQ, w), 1)
                diff = iq + (b * BQ - lo) - ik
                s = jnp.where(
                    (diff >= -WINDOW) & (diff <= WINDOW), s, -1e9
                )
                m = jnp.max(s, axis=-1, keepdims=True)
                e = jnp.exp(s - m)
                den = jnp.sum(e, axis=-1, keepdims=True)
                wgt = (e / den).astype(jnp.bfloat16)
                vwin = v_ref[h, pl.ds(lo, w), :]
                ctxb = lax.dot_general(
                    wgt, vwin, (((1,), (0,)), ((), ())),
                    preferred_element_type=jnp.float32,
                )
                ctx_buf[pl.ds(b * BQ, BQ), pl.ds(h * Dh, Dh)] = (
                    ctxb.astype(jnp.bfloat16))

        h_own = [jnp.where((p == 1) | (p == 2), 1, 0), jnp.where(p >= 2, 1, 0)]
        q2nd = [jnp.where(p >= 2, 1, 0), lax.rem(p, 2)]
        partners = [(q1, q2), (q2, q1)]

        def exchange(src, dst, sem_idx, tgt):
            rdma = pltpu.make_async_remote_copy(
                src_ref=src, dst_ref=dst,
                send_sem=send_sems.at[sem_idx],
                recv_sem=recv_sems.at[sem_idx],
                device_id=(tgt,), device_id_type=pl.DeviceIdType.MESH,
            )
            rdma.start()
            return rdma

        rds = []
        for s in range(2):
            wo_s = wo_ref[:, pl.ds(s * HC, HC)]
            s1_send[s, :, :] = jnp.dot(
                ctx_buf[pl.ds((1 - h_own[s]) * HR, HR), :], wo_s,
                preferred_element_type=jnp.float32,
            ).astype(jnp.bfloat16)
            rds.append(exchange(s1_send.at[s], s1_recv.at[s], s,
                                partners[s][0]))
            part_own[s, :, :] = jnp.dot(
                ctx_buf[pl.ds(h_own[s] * HR, HR), :], wo_s,
                preferred_element_type=jnp.float32,
            ).astype(jnp.bfloat16)
        for r in rds:
            r.wait()

        rds = []
        for s in range(2):
            off_other = (1 - q2nd[s]) * QR
            off_own = q2nd[s] * QR
            s2_send[s, :, :] = (
                part_own[s, pl.ds(off_other, QR), :].astype(jnp.float32)
                + s1_recv[s, pl.ds(off_other, QR), :].astype(jnp.float32)
            ).astype(jnp.bfloat16)
            qtr_f32[s, :, :] = (
                part_own[s, pl.ds(off_own, QR), :].astype(jnp.float32)
                + s1_recv[s, pl.ds(off_own, QR), :].astype(jnp.float32)
            )
            rds.append(exchange(s2_send.at[s], s2_recv.at[s], 2 + s,
                                partners[s][1]))
        for r in rds:
            r.wait()

        rds = []
        for s in range(2):
            rq = 2 * h_own[s] + q2nd[s]
            reduced = qtr_f32[s, :, :] + s2_recv[s, :, :].astype(jnp.float32)
            out_ref[pl.ds(rq * QR, QR), pl.ds(s * HC, HC)] = reduced
            s3_send[s, :, :] = reduced.astype(jnp.bfloat16)
            rds.append(exchange(s3_send.at[s], s3_recv.at[s], 4 + s,
                                partners[s][1]))
        for r in rds:
            r.wait()

        rds = []
        for s in range(2):
            rq2 = 2 * h_own[s] + (1 - q2nd[s])
            out_ref[pl.ds(rq2 * QR, QR), pl.ds(s * HC, HC)] = (
                s3_recv[s, :, :].astype(jnp.float32))
            s4_send[s, pl.ds(q2nd[s] * QR, QR), :] = s3_send[s, :, :]
            s4_send[s, pl.ds((1 - q2nd[s]) * QR, QR), :] = s3_recv[s, :, :]
            rds.append(exchange(s4_send.at[s], s4_recv.at[s], 6 + s,
                                partners[s][0]))
        for r in rds:
            r.wait()
        for s in range(2):
            out_ref[pl.ds((1 - h_own[s]) * HR, HR), pl.ds(s * HC, HC)] = (
                s4_recv[s, :, :].astype(jnp.float32))

    out = pl.pallas_call(
        body,
        out_shape=jax.ShapeDtypeStruct((M, N), jnp.float32),
        in_specs=[pl.BlockSpec(memory_space=pltpu.VMEM)] * 5,
        out_specs=pl.BlockSpec(memory_space=pltpu.VMEM),
        scratch_shapes=[
            pltpu.VMEM((M, hd), jnp.bfloat16),
            pltpu.VMEM((M, hd), jnp.bfloat16),
            pltpu.VMEM((2, HR, HC), jnp.bfloat16),
            pltpu.VMEM((2, HR, HC), jnp.bfloat16),
            pltpu.VMEM((2, HR, HC), jnp.bfloat16),
            pltpu.VMEM((2, QR, HC), jnp.bfloat16),
            pltpu.VMEM((2, QR, HC), jnp.bfloat16),
            pltpu.VMEM((2, QR, HC), jnp.bfloat16),
            pltpu.VMEM((2, QR, HC), jnp.bfloat16),
            pltpu.VMEM((2, HR, HC), jnp.bfloat16),
            pltpu.VMEM((2, HR, HC), jnp.bfloat16),
            pltpu.VMEM((2, QR, HC), jnp.float32),
            pltpu.SemaphoreType.DMA((8,)),
            pltpu.SemaphoreType.DMA((8,)),
        ],
        compiler_params=pltpu.CompilerParams(collective_id=0),
    )(xb, Wq_l, Kh, Vh, Wo_l)
    return out[None, :, :]


# device time: 65559 ns/iter; 1.5647x vs baseline; 1.0256x over previous
import jax
import jax.numpy as jnp
from jax import lax
from jax.experimental import pallas as pl
from jax.experimental.pallas import tpu as pltpu

N_DEV = 4
SCALE = 0.08838834764831843
WINDOW = 128
BQ = 128


def kernel(x, Wq, K_ext, V_ext, Wo):
    my = lax.axis_index("i")
    B, Sq, D = x.shape
    _, Skv, Hl, Dh = K_ext.shape
    hd = Hl * Dh
    start = my * hd

    Wq_l = lax.dynamic_slice_in_dim(Wq, start, hd, axis=1).astype(jnp.bfloat16)
    Wo_l = lax.dynamic_slice_in_dim(Wo, start, hd, axis=0).astype(jnp.bfloat16)
    Kf = K_ext[0].reshape(Skv, hd)
    Vf = V_ext[0].reshape(Skv, hd)

    M, N = Sq, D
    HC = N // 2
    HR = M // 2
    QR = M // 4
    nb = Sq // BQ

    def body(x_ref, wq_ref, k_ref, v_ref, wo_ref, out_ref,
             q_buf, ctx_buf, kb_buf, vb_buf, part_own,
             s1_send, s1_recv, s2_send, s2_recv,
             s3_send, s3_recv, s4_send, s4_recv,
             qtr_f32, send_sems, recv_sems):
        p = lax.axis_index("i")
        q1 = jnp.bitwise_xor(p, 1)
        q2 = 3 - p

        barrier_sem = pltpu.get_barrier_semaphore()
        for nbr in (q1, q2):
            pl.semaphore_signal(
                barrier_sem, inc=1,
                device_id=(nbr,), device_id_type=pl.DeviceIdType.MESH,
            )
        pl.semaphore_wait(barrier_sem, 2)

        kb_buf[:, :] = k_ref[:, :].astype(jnp.bfloat16)
        vb_buf[:, :] = v_ref[:, :].astype(jnp.bfloat16)
        q_buf[:, :] = jnp.dot(
            x_ref[:, :].astype(jnp.bfloat16), wq_ref[:, :],
            preferred_element_type=jnp.float32,
        ).astype(jnp.bfloat16)

        for h in range(Hl):
            for b in range(nb):
                lo = max(0, b * BQ - WINDOW)
                hi = min(Skv, b * BQ + BQ + WINDOW)
                w = hi - lo
                qb = q_buf[pl.ds(b * BQ, BQ), pl.ds(h * Dh, Dh)]
                kwin = kb_buf[pl.ds(lo, w), pl.ds(h * Dh, Dh)]
                s = lax.dot_general(
                    qb, kwin, (((1,), (1,)), ((), ())),
                    preferred_element_type=jnp.float32,
                ) * SCALE
                iq = lax.broadcasted_iota(jnp.int32, (BQ, w), 0)
                ik = lax.broadcasted_iota(jnp.int32, (BQ, w), 1)
                diff = iq + (b * BQ - lo) - ik
                s = jnp.where(
                    (diff >= -WINDOW) & (diff <= WINDOW), s, -1e9
                )
                m = jnp.max(s, axis=-1, keepdims=True)
                e = jnp.exp(s - m)
                den = jnp.sum(e, axis=-1, keepdims=True)
                wgt = (e / den).astype(jnp.bfloat16)
                vwin = vb_buf[pl.ds(lo, w), pl.ds(h * Dh, Dh)]
                ctxb = lax.dot_general(
                    wgt, vwin, (((1,), (0,)), ((), ())),
                    preferred_element_type=jnp.float32,
                )
                ctx_buf[pl.ds(b * BQ, BQ), pl.ds(h * Dh, Dh)] = (
                    ctxb.astype(jnp.bfloat16))

        h_own = [jnp.where((p == 1) | (p == 2), 1, 0), jnp.where(p >= 2, 1, 0)]
        q2nd = [jnp.where(p >= 2, 1, 0), lax.rem(p, 2)]
        partners = [(q1, q2), (q2, q1)]

        def exchange(src, dst, sem_idx, tgt):
            rdma = pltpu.make_async_remote_copy(
                src_ref=src, dst_ref=dst,
                send_sem=send_sems.at[sem_idx],
                recv_sem=recv_sems.at[sem_idx],
                device_id=(tgt,), device_id_type=pl.DeviceIdType.MESH,
            )
            rdma.start()
            return rdma

        rds = []
        for s in range(2):
            wo_s = wo_ref[:, pl.ds(s * HC, HC)]
            s1_send[s, :, :] = jnp.dot(
                ctx_buf[pl.ds((1 - h_own[s]) * HR, HR), :], wo_s,
                preferred_element_type=jnp.float32,
            ).astype(jnp.bfloat16)
            rds.append(exchange(s1_send.at[s], s1_recv.at[s], s,
                                partners[s][0]))
            part_own[s, :, :] = jnp.dot(
                ctx_buf[pl.ds(h_own[s] * HR, HR), :], wo_s,
                preferred_element_type=jnp.float32,
            ).astype(jnp.bfloat16)
        for r in rds:
            r.wait()

        rds = []
        for s in range(2):
            off_other = (1 - q2nd[s]) * QR
            off_own = q2nd[s] * QR
            s2_send[s, :, :] = (
                part_own[s, pl.ds(off_other, QR), :].astype(jnp.float32)
                + s1_recv[s, pl.ds(off_other, QR), :].astype(jnp.float32)
            ).astype(jnp.bfloat16)
            qtr_f32[s, :, :] = (
                part_own[s, pl.ds(off_own, QR), :].astype(jnp.float32)
                + s1_recv[s, pl.ds(off_own, QR), :].astype(jnp.float32)
            )
            rds.append(exchange(s2_send.at[s], s2_recv.at[s], 2 + s,
                                partners[s][1]))
        for r in rds:
            r.wait()

        rds = []
        for s in range(2):
            rq = 2 * h_own[s] + q2nd[s]
            reduced = qtr_f32[s, :, :] + s2_recv[s, :, :].astype(jnp.float32)
            out_ref[pl.ds(rq * QR, QR), pl.ds(s * HC, HC)] = reduced
            s3_send[s, :, :] = reduced.astype(jnp.bfloat16)
            rds.append(exchange(s3_send.at[s], s3_recv.at[s], 4 + s,
                                partners[s][1]))
        for r in rds:
            r.wait()

        rds = []
        for s in range(2):
            rq2 = 2 * h_own[s] + (1 - q2nd[s])
            out_ref[pl.ds(rq2 * QR, QR), pl.ds(s * HC, HC)] = (
                s3_recv[s, :, :].astype(jnp.float32))
            s4_send[s, pl.ds(q2nd[s] * QR, QR), :] = s3_send[s, :, :]
            s4_send[s, pl.ds((1 - q2nd[s]) * QR, QR), :] = s3_recv[s, :, :]
            rds.append(exchange(s4_send.at[s], s4_recv.at[s], 6 + s,
                                partners[s][0]))
        for r in rds:
            r.wait()
        for s in range(2):
            out_ref[pl.ds((1 - h_own[s]) * HR, HR), pl.ds(s * HC, HC)] = (
                s4_recv[s, :, :].astype(jnp.float32))

    out = pl.pallas_call(
        body,
        out_shape=jax.ShapeDtypeStruct((M, N), jnp.float32),
        in_specs=[pl.BlockSpec(memory_space=pltpu.VMEM)] * 5,
        out_specs=pl.BlockSpec(memory_space=pltpu.VMEM),
        scratch_shapes=[
            pltpu.VMEM((M, hd), jnp.bfloat16),
            pltpu.VMEM((M, hd), jnp.bfloat16),
            pltpu.VMEM((Skv, hd), jnp.bfloat16),
            pltpu.VMEM((Skv, hd), jnp.bfloat16),
            pltpu.VMEM((2, HR, HC), jnp.bfloat16),
            pltpu.VMEM((2, HR, HC), jnp.bfloat16),
            pltpu.VMEM((2, HR, HC), jnp.bfloat16),
            pltpu.VMEM((2, QR, HC), jnp.bfloat16),
            pltpu.VMEM((2, QR, HC), jnp.bfloat16),
            pltpu.VMEM((2, QR, HC), jnp.bfloat16),
            pltpu.VMEM((2, QR, HC), jnp.bfloat16),
            pltpu.VMEM((2, HR, HC), jnp.bfloat16),
            pltpu.VMEM((2, HR, HC), jnp.bfloat16),
            pltpu.VMEM((2, QR, HC), jnp.float32),
            pltpu.SemaphoreType.DMA((8,)),
            pltpu.SemaphoreType.DMA((8,)),
        ],
        compiler_params=pltpu.CompilerParams(collective_id=0),
    )(x[0], Wq_l, Kf, Vf, Wo_l)
    return out[None, :, :]


# device time: 64002 ns/iter; 1.6027x vs baseline; 1.0243x over previous
import jax
import jax.numpy as jnp
from jax import lax
from jax.experimental import pallas as pl
from jax.experimental.pallas import tpu as pltpu

N_DEV = 4
SCALE = 0.08838834764831843
WINDOW = 128
BQ = 128


def kernel(x, Wq, K_ext, V_ext, Wo):
    my = lax.axis_index("i")
    B, Sq, D = x.shape
    _, Skv, Hl, Dh = K_ext.shape
    hd = Hl * Dh
    start = my * hd

    M, N = Sq, D
    HC = N // 2
    HR = M // 2
    QR = M // 4
    nb = Sq // BQ

    def body(x_ref, wq_ref, k_ref, v_ref, wo_ref, out_ref,
             q_buf, ctx_buf, wq_f32, wo_f32, kv_f32,
             wq_b, wo_b, kb_buf, vb_buf, part_own,
             s1_send, s1_recv, s2_send, s2_recv,
             s3_recv, s4_recv,
             qtr_bf, prep_sems, send_sems, recv_sems):
        s3_send = s2_send
        s4_send = s1_send
        p = lax.axis_index("i")
        q1 = jnp.bitwise_xor(p, 1)
        q2 = 3 - p

        kv_cps = []
        for h in range(Hl):
            for t, ref in ((0, k_ref), (1, v_ref)):
                cp = pltpu.make_async_copy(
                    ref.at[:, h, :], kv_f32.at[t, h],
                    prep_sems.at[2 + 2 * h + t],
                )
                cp.start()
                kv_cps.append(cp)
        wq_cp = pltpu.make_async_copy(
            wq_ref.at[:, pl.ds(p * hd, hd)], wq_f32, prep_sems.at[0])
        wq_cp.start()
        wo_cp = pltpu.make_async_copy(
            wo_ref.at[pl.ds(p * hd, hd), :], wo_f32, prep_sems.at[1])
        wo_cp.start()

        barrier_sem = pltpu.get_barrier_semaphore()
        for nbr in (q1, q2):
            pl.semaphore_signal(
                barrier_sem, inc=1,
                device_id=(nbr,), device_id_type=pl.DeviceIdType.MESH,
            )
        pl.semaphore_wait(barrier_sem, 2)

        wq_cp.wait()
        wq_b[:, :] = wq_f32[:, :].astype(jnp.bfloat16)
        q_buf[:, :] = jnp.dot(
            x_ref[:, :].astype(jnp.bfloat16), wq_b[:, :],
            preferred_element_type=jnp.float32,
        ).astype(jnp.bfloat16)
        for cp in kv_cps:
            cp.wait()
        kb_buf[:, :, :] = kv_f32[0].astype(jnp.bfloat16)
        vb_buf[:, :, :] = kv_f32[1].astype(jnp.bfloat16)

        for h in range(Hl):
            for b in range(nb):
                lo = max(0, b * BQ - WINDOW)
                hi = min(Skv, b * BQ + BQ + WINDOW)
                w = hi - lo
                qb = q_buf[pl.ds(b * BQ, BQ), pl.ds(h * Dh, Dh)]
                kwin = kb_buf[h, pl.ds(lo, w), :]
                s = lax.dot_general(
                    qb, kwin, (((1,), (1,)), ((), ())),
                    preferred_element_type=jnp.float32,
                ) * SCALE
                iq = lax.broadcasted_iota(jnp.int32, (BQ, w), 0)
                ik = lax.broadcasted_iota(jnp.int32, (BQ, w), 1)
                diff = iq + (b * BQ - lo) - ik
                s = jnp.where(
                    (diff >= -WINDOW) & (diff <= WINDOW), s, -1e9
                )
                m = jnp.max(s, axis=-1, keepdims=True)
                e = jnp.exp(s - m)
                den = jnp.sum(e, axis=-1, keepdims=True)
                wgt = (e / den).astype(jnp.bfloat16)
                vwin = vb_buf[h, pl.ds(lo, w), :]
                ctxb = lax.dot_general(
                    wgt, vwin, (((1,), (0,)), ((), ())),
                    preferred_element_type=jnp.float32,
                )
                ctx_buf[pl.ds(b * BQ, BQ), pl.ds(h * Dh, Dh)] = (
                    ctxb.astype(jnp.bfloat16))

        h_own = [jnp.where((p == 1) | (p == 2), 1, 0), jnp.where(p >= 2, 1, 0)]
        q2nd = [jnp.where(p >= 2, 1, 0), lax.rem(p, 2)]
        partners = [(q1, q2), (q2, q1)]

        def exchange(src, dst, sem_idx, tgt):
            rdma = pltpu.make_async_remote_copy(
                src_ref=src, dst_ref=dst,
                send_sem=send_sems.at[sem_idx],
                recv_sem=recv_sems.at[sem_idx],
                device_id=(tgt,), device_id_type=pl.DeviceIdType.MESH,
            )
            rdma.start()
            return rdma

        wo_cp.wait()
        wo_b[:, :] = wo_f32[:, :].astype(jnp.bfloat16)
        rds = []
        for s in range(2):
            wo_s = wo_b[:, pl.ds(s * HC, HC)]
            s1_send[s, :, :] = jnp.dot(
                ctx_buf[pl.ds((1 - h_own[s]) * HR, HR), :], wo_s,
                preferred_element_type=jnp.float32,
            ).astype(jnp.bfloat16)
            rds.append(exchange(s1_send.at[s], s1_recv.at[s], s,
                                partners[s][0]))
            part_own[s, :, :] = jnp.dot(
                ctx_buf[pl.ds(h_own[s] * HR, HR), :], wo_s,
                preferred_element_type=jnp.float32,
            ).astype(jnp.bfloat16)
        for r in rds:
            r.wait()

        rds = []
        for s in range(2):
            off_other = (1 - q2nd[s]) * QR
            off_own = q2nd[s] * QR
            s2_send[s, :, :] = (
                part_own[s, pl.ds(off_other, QR), :].astype(jnp.float32)
                + s1_recv[s, pl.ds(off_other, QR), :].astype(jnp.float32)
            ).astype(jnp.bfloat16)
            qtr_bf[s, :, :] = (
                part_own[s, pl.ds(off_own, QR), :].astype(jnp.float32)
                + s1_recv[s, pl.ds(off_own, QR), :].astype(jnp.float32)
            ).astype(jnp.bfloat16)
            rds.append(exchange(s2_send.at[s], s2_recv.at[s], 2 + s,
                                partners[s][1]))
        for r in rds:
            r.wait()

        rds = []
        for s in range(2):
            rq = 2 * h_own[s] + q2nd[s]
            reduced = (qtr_bf[s, :, :].astype(jnp.float32)
                       + s2_recv[s, :, :].astype(jnp.float32))
            out_ref[pl.ds(rq * QR, QR), pl.ds(s * HC, HC)] = reduced
            s3_send[s, :, :] = reduced.astype(jnp.bfloat16)
            rds.append(exchange(s3_send.at[s], s3_recv.at[s], 4 + s,
                                partners[s][1]))
        for r in rds:
            r.wait()

        rds = []
        for s in range(2):
            rq2 = 2 * h_own[s] + (1 - q2nd[s])
            out_ref[pl.ds(rq2 * QR, QR), pl.ds(s * HC, HC)] = (
                s3_recv[s, :, :].astype(jnp.float32))
            s4_send[s, pl.ds(q2nd[s] * QR, QR), :] = s3_send[s, :, :]
            s4_send[s, pl.ds((1 - q2nd[s]) * QR, QR), :] = s3_recv[s, :, :]
            rds.append(exchange(s4_send.at[s], s4_recv.at[s], 6 + s,
                                partners[s][0]))
        for r in rds:
            r.wait()
        for s in range(2):
            out_ref[pl.ds((1 - h_own[s]) * HR, HR), pl.ds(s * HC, HC)] = (
                s4_recv[s, :, :].astype(jnp.float32))

    out = pl.pallas_call(
        body,
        out_shape=jax.ShapeDtypeStruct((M, N), jnp.float32),
        in_specs=[pl.BlockSpec(memory_space=pltpu.VMEM)]
        + [pl.BlockSpec(memory_space=pl.ANY)] * 4,
        out_specs=pl.BlockSpec(memory_space=pltpu.VMEM),
        scratch_shapes=[
            pltpu.VMEM((M, hd), jnp.bfloat16),
            pltpu.VMEM((M, hd), jnp.bfloat16),
            pltpu.VMEM((D, hd), jnp.float32),
            pltpu.VMEM((hd, N), jnp.float32),
            pltpu.VMEM((2, Hl, Skv, Dh), jnp.float32),
            pltpu.VMEM((D, hd), jnp.bfloat16),
            pltpu.VMEM((hd, N), jnp.bfloat16),
            pltpu.VMEM((Hl, Skv, Dh), jnp.bfloat16),
            pltpu.VMEM((Hl, Skv, Dh), jnp.bfloat16),
            pltpu.VMEM((2, HR, HC), jnp.bfloat16),
            pltpu.VMEM((2, HR, HC), jnp.bfloat16),
            pltpu.VMEM((2, HR, HC), jnp.bfloat16),
            pltpu.VMEM((2, QR, HC), jnp.bfloat16),
            pltpu.VMEM((2, QR, HC), jnp.bfloat16),
            pltpu.VMEM((2, QR, HC), jnp.bfloat16),
            pltpu.VMEM((2, HR, HC), jnp.bfloat16),
            pltpu.VMEM((2, QR, HC), jnp.bfloat16),
            pltpu.SemaphoreType.DMA((2 + 2 * Hl,)),
            pltpu.SemaphoreType.DMA((8,)),
            pltpu.SemaphoreType.DMA((8,)),
        ],
        compiler_params=pltpu.CompilerParams(
            collective_id=0, vmem_limit_bytes=48 * 1024 * 1024,
        ),
    )(x[0], Wq, K_ext[0], V_ext[0], Wo)
    return out[None, :, :]


# device time: 38603 ns/iter; 2.6573x vs baseline; 1.6580x over previous
import jax
import jax.numpy as jnp
from jax import lax
from jax.experimental import pallas as pl
from jax.experimental.pallas import tpu as pltpu

SCALE = 0.08838834764831843
WINDOW = 128
BQ = 128


def kernel(x, Wq, K_ext, V_ext, Wo):
    B, Sq, D = x.shape
    _, Skv, Hl, Dh = K_ext.shape
    hd = Hl * Dh

    M, N = Sq, D
    HC = N // 2
    HR = M // 2
    QR = M // 4
    nb = Sq // BQ

    def body(x_ref, wq_ref, k_ref, v_ref, wo_ref, out_ref,
             q_buf, ctx_buf, wq_f32, wo_f32, kv_f32,
             wq_b, wo_b, kb_buf, vb_buf, bias_buf, part_own,
             t1_send, t1_recv, t2_send, t2_recv, t3_recv,
             prep_sems, send_sems, recv_sems):
        p = lax.axis_index("i")
        q1 = jnp.bitwise_xor(p, 1)
        q2 = 3 - p

        wq_cp = pltpu.make_async_copy(
            wq_ref.at[:, pl.ds(p * hd, hd)], wq_f32, prep_sems.at[0])
        wq_cp.start()
        kv_cps = []
        for h in range(Hl):
            for t, ref in ((0, k_ref), (1, v_ref)):
                cp = pltpu.make_async_copy(
                    ref.at[:, h, :], kv_f32.at[t, h],
                    prep_sems.at[2 + 2 * h + t],
                )
                cp.start()
                kv_cps.append(cp)
        wo_cp = pltpu.make_async_copy(
            wo_ref.at[pl.ds(p * hd, hd), :], wo_f32, prep_sems.at[1])
        wo_cp.start()

        barrier_sem = pltpu.get_barrier_semaphore()
        for nbr in (q1, q2):
            pl.semaphore_signal(
                barrier_sem, inc=1,
                device_id=(nbr,), device_id_type=pl.DeviceIdType.MESH,
            )
        pl.semaphore_wait(barrier_sem, 2)

        wq_cp.wait()
        wq_b[:, :] = wq_f32[:, :].astype(jnp.bfloat16)
        q_buf[:, :] = (jnp.dot(
            x_ref[:, :].astype(jnp.bfloat16), wq_b[:, :],
            preferred_element_type=jnp.float32,
        ) * SCALE).astype(jnp.bfloat16)
        for cp in kv_cps:
            cp.wait()
        kb_buf[:, :, :] = kv_f32[0].astype(jnp.bfloat16)
        vb_buf[:, :, :Dh] = kv_f32[1].astype(jnp.bfloat16)
        vb_buf[:, :, Dh:] = jnp.ones((Hl, Skv, Dh), jnp.bfloat16)

        iq = lax.broadcasted_iota(jnp.int32, (BQ, 3 * BQ), 0)
        ik = lax.broadcasted_iota(jnp.int32, (BQ, 3 * BQ), 1)
        diff = iq + BQ - ik
        bias_buf[:, :] = jnp.where(
            (diff >= -WINDOW) & (diff <= WINDOW), 0.0, -1e9)

        for h in range(Hl):
            for b in range(nb):
                lo = max(0, b * BQ - WINDOW)
                hi = min(Skv, b * BQ + BQ + WINDOW)
                w = hi - lo
                coloff = lo - (b - 1) * BQ
                qb = q_buf[pl.ds(b * BQ, BQ), pl.ds(h * Dh, Dh)]
                kwin = kb_buf[h, pl.ds(lo, w), :]
                s = lax.dot_general(
                    qb, kwin, (((1,), (1,)), ((), ())),
                    preferred_element_type=jnp.float32,
                )
                wgt = jnp.exp(
                    s + bias_buf[:, pl.ds(coloff, w)]).astype(jnp.bfloat16)
                vwin = vb_buf[h, pl.ds(lo, w), :Dh + 1]
                cd = lax.dot_general(
                    wgt, vwin, (((1,), (0,)), ((), ())),
                    preferred_element_type=jnp.float32,
                )
                ctx_buf[pl.ds(b * BQ, BQ), pl.ds(h * Dh, Dh)] = (
                    cd[:, :Dh] * (1.0 / cd[:, Dh:])).astype(jnp.bfloat16)

        h_own = [jnp.where((p == 1) | (p == 2), 1, 0), jnp.where(p >= 2, 1, 0)]
        partners = [(q1, q2), (q2, q1)]

        def exchange(src, dst, sem_idx, tgt):
            rdma = pltpu.make_async_remote_copy(
                src_ref=src, dst_ref=dst,
                send_sem=send_sems.at[sem_idx],
                recv_sem=recv_sems.at[sem_idx],
                device_id=(tgt,), device_id_type=pl.DeviceIdType.MESH,
            )
            rdma.start()
            return rdma

        wo_cp.wait()
        wo_b[:, :] = wo_f32[:, :].astype(jnp.bfloat16)
        rd1, rd2, rd3 = {}, {}, {}
        for s in range(2):
            wo_s = wo_b[:, pl.ds(s * HC, HC)]
            t1_send[s, :, :] = jnp.dot(
                ctx_buf[pl.ds((1 - h_own[s]) * HR, HR), :], wo_s,
                preferred_element_type=jnp.float32,
            ).astype(jnp.bfloat16)
            for c in range(2):
                rd1[s, c] = exchange(
                    t1_send.at[s, pl.ds(c * QR, QR)],
                    t1_recv.at[s, pl.ds(c * QR, QR)],
                    s * 2 + c, partners[s][0])
            part_own[s, :, :] = jnp.dot(
                ctx_buf[pl.ds(h_own[s] * HR, HR), :], wo_s,
                preferred_element_type=jnp.float32,
            ).astype(jnp.bfloat16)

        for c in range(2):
            for s in range(2):
                rd1[s, c].wait()
                t2_send[s, pl.ds(c * QR, QR), :] = (
                    part_own[s, pl.ds(c * QR, QR), :].astype(jnp.float32)
                    + t1_recv[s, pl.ds(c * QR, QR), :].astype(jnp.float32)
                ).astype(jnp.bfloat16)
                rd2[s, c] = exchange(
                    t2_send.at[s, pl.ds(c * QR, QR)],
                    t2_recv.at[s, pl.ds(c * QR, QR)],
                    4 + s * 2 + c, partners[s][1])

        for c in range(2):
            for s in range(2):
                rd2[s, c].wait()
                red = (t2_send[s, pl.ds(c * QR, QR), :].astype(jnp.float32)
                       + t2_recv[s, pl.ds(c * QR, QR), :].astype(jnp.float32)
                       ).astype(jnp.bfloat16)
                out_ref[pl.ds(h_own[s] * HR + c * QR, QR),
                        pl.ds(s * HC, HC)] = red
                t1_send[s, pl.ds(c * QR, QR), :] = red
                rd3[s, c] = exchange(
                    t1_send.at[s, pl.ds(c * QR, QR)],
                    t3_recv.at[s, pl.ds(c * QR, QR)],
                    8 + s * 2 + c, partners[s][0])
        for c in range(2):
            for s in range(2):
                rd3[s, c].wait()
                out_ref[pl.ds((1 - h_own[s]) * HR + c * QR, QR),
                        pl.ds(s * HC, HC)] = t3_recv[s, pl.ds(c * QR, QR), :]

    out = pl.pallas_call(
        body,
        out_shape=jax.ShapeDtypeStruct((M, N), jnp.bfloat16),
        in_specs=[pl.BlockSpec(memory_space=pltpu.VMEM)]
        + [pl.BlockSpec(memory_space=pl.ANY)] * 4,
        out_specs=pl.BlockSpec(memory_space=pltpu.VMEM),
        scratch_shapes=[
            pltpu.VMEM((M, hd), jnp.bfloat16),
            pltpu.VMEM((M, hd), jnp.bfloat16),
            pltpu.VMEM((D, hd), jnp.float32),
            pltpu.VMEM((hd, N), jnp.float32),
            pltpu.VMEM((2, Hl, Skv, Dh), jnp.float32),
            pltpu.VMEM((D, hd), jnp.bfloat16),
            pltpu.VMEM((hd, N), jnp.bfloat16),
            pltpu.VMEM((Hl, Skv, Dh), jnp.bfloat16),
            pltpu.VMEM((Hl, Skv, 2 * Dh), jnp.bfloat16),
            pltpu.VMEM((BQ, 3 * BQ), jnp.float32),
            pltpu.VMEM((2, HR, HC), jnp.bfloat16),
            pltpu.VMEM((2, HR, HC), jnp.bfloat16),
            pltpu.VMEM((2, HR, HC), jnp.bfloat16),
            pltpu.VMEM((2, HR, HC), jnp.bfloat16),
            pltpu.VMEM((2, HR, HC), jnp.bfloat16),
            pltpu.VMEM((2, HR, HC), jnp.bfloat16),
            pltpu.SemaphoreType.DMA((2 + 2 * Hl,)),
            pltpu.SemaphoreType.DMA((12,)),
            pltpu.SemaphoreType.DMA((12,)),
        ],
        compiler_params=pltpu.CompilerParams(
            collective_id=0, vmem_limit_bytes=48 * 1024 * 1024,
        ),
    )(x[0], Wq, K_ext[0], V_ext[0], Wo)
    return out[None, :, :]
